# baseline (speedup 1.0000x reference)
import functools

import jax
import jax.numpy as jnp
from jax import lax
from jax.experimental import pallas as pl
from jax.experimental.pallas import tpu as pltpu
from jax.experimental.pallas import tpu_sc as plsc

NC = 2
NS = 16
NW = NC * NS
D = 64


@functools.lru_cache(maxsize=None)
def _make_kernel(B: int, C: int):
    n_chunks = B // (NW * C)
    mesh = plsc.VectorSubcoreMesh(core_axis_name="c", subcore_axis_name="s")

    @functools.partial(
        pl.kernel,
        out_type=jax.ShapeDtypeStruct((B, D), jnp.float32),
        mesh=mesh,
        compiler_params=pltpu.CompilerParams(use_tc_tiling_on_sc=False),
        scratch_types=[
            pltpu.VMEM((n_chunks, C), jnp.int32),
            pltpu.VMEM((C, D), jnp.float32),
            pltpu.SemaphoreType.DMA,
        ],
    )
    def k(ids_hbm, w_hbm, out_hbm, idx_v, rows_v, sem):
        wid = lax.axis_index("s") * NC + lax.axis_index("c")
        base = wid * (n_chunks * C)
        pltpu.sync_copy(ids_hbm.at[wid], idx_v)

        @pl.loop(0, n_chunks)
        def _chunk(kk):
            pltpu.async_copy(w_hbm.at[idx_v.at[kk]], rows_v, sem).wait()
            pltpu.sync_copy(rows_v, out_hbm.at[pl.ds(base + kk * C, C)])

    return k


@jax.jit
def kernel(ids, weight):
    S0, S1 = ids.shape
    B = S0 * S1
    C = 128
    ids3 = ids.astype(jnp.int32).reshape(NW, B // (NW * C), C)
    out = _make_kernel(B, C)(ids3, weight)
    return out.reshape(S0, S1, D)


# speedup vs baseline: 1.1128x; 1.1128x over previous
import functools

import jax
import jax.numpy as jnp
from jax import lax
from jax.experimental import pallas as pl
from jax.experimental.pallas import tpu as pltpu
from jax.experimental.pallas import tpu_sc as plsc

NC = 2
NS = 16
NW = NC * NS
D = 64
C = 128
K = 4
CS = C * K


@functools.lru_cache(maxsize=None)
def _make_kernel(B: int):
    n_chunks = B // (NW * C)
    n_super = B // (NW * CS)
    mesh = plsc.VectorSubcoreMesh(core_axis_name="c", subcore_axis_name="s")

    @functools.partial(
        pl.kernel,
        out_type=jax.ShapeDtypeStruct((B, D), jnp.float32),
        mesh=mesh,
        compiler_params=pltpu.CompilerParams(use_tc_tiling_on_sc=False),
        scratch_types=[
            pltpu.VMEM((n_chunks, C), jnp.int32),
            pltpu.VMEM((2, CS, D), jnp.float32),
            pltpu.SemaphoreType.DMA((2,)),
            pltpu.SemaphoreType.DMA((2,)),
        ],
    )
    def k(ids_hbm, w_hbm, out_hbm, idx_v, rows_v, gsem, osem):
        wid = lax.axis_index("s") * NC + lax.axis_index("c")
        base = wid * (n_super * CS)
        pltpu.sync_copy(ids_hbm.at[wid], idx_v)

        def fire_gathers(s, b):
            for j in range(K):
                pltpu.async_copy(
                    w_hbm.at[idx_v.at[s * K + j]],
                    rows_v.at[b, pl.ds(j * C, C)],
                    gsem.at[b],
                )

        def wait_gathers(b):
            pltpu.make_async_copy(
                out_hbm.at[pl.ds(0, CS)], rows_v.at[b], gsem.at[b]
            ).wait()

        def start_out(s, b):
            pltpu.async_copy(
                rows_v.at[b], out_hbm.at[pl.ds(base + s * CS, CS)], osem.at[b]
            )

        def wait_out(b):
            pltpu.make_async_copy(
                rows_v.at[b], out_hbm.at[pl.ds(0, CS)], osem.at[b]
            ).wait()

        fire_gathers(0, 0)

        @pl.loop(0, n_super)
        def _super(g):
            b = g % 2
            nb = (g + 1) % 2

            @pl.when(g + 1 < n_super)
            def _fire_next():
                @pl.when(g >= 1)
                def _recycle():
                    wait_out(nb)

                fire_gathers(g + 1, nb)

            wait_gathers(b)
            start_out(g, b)

        wait_out((n_super - 1) % 2)

    return k


@jax.jit
def kernel(ids, weight):
    S0, S1 = ids.shape
    B = S0 * S1
    ids3 = ids.astype(jnp.int32).reshape(NW, B // (NW * C), C)
    out = _make_kernel(B)(ids3, weight)
    return out.reshape(S0, S1, D)


# speedup vs baseline: 1.4802x; 1.3301x over previous
import functools

import jax
import jax.numpy as jnp
from jax import lax
from jax.experimental import pallas as pl
from jax.experimental.pallas import tpu as pltpu
from jax.experimental.pallas import tpu_sc as plsc

NC = 2
NS = 16
NW = NC * NS
D = 64
C = 128
K = 4
CS = C * K


@functools.lru_cache(maxsize=None)
def _make_kernel(B: int):
    n_chunks = B // (NW * C)
    n_super = B // (NW * CS)
    mesh = plsc.VectorSubcoreMesh(core_axis_name="c", subcore_axis_name="s")

    @functools.partial(
        pl.kernel,
        out_type=jax.ShapeDtypeStruct((B, 2 * D), jnp.float32),
        mesh=mesh,
        compiler_params=pltpu.CompilerParams(use_tc_tiling_on_sc=False),
        scratch_types=[
            pltpu.VMEM((n_chunks, C), jnp.int32),
            pltpu.VMEM((2, CS, D), jnp.float32),
            pltpu.SemaphoreType.DMA((2,)),
            pltpu.SemaphoreType.DMA((2,)),
        ],
    )
    def k(ids_hbm, w_hbm, out_hbm, idx_v, rows_v, gsem, osem):
        wid = lax.axis_index("s") * NC + lax.axis_index("c")
        base = wid * (n_super * CS)
        pltpu.sync_copy(ids_hbm.at[wid], idx_v)

        def fire_gathers(s, b):
            for j in range(K):
                pltpu.async_copy(
                    w_hbm.at[idx_v.at[s * K + j]],
                    rows_v.at[b, pl.ds(j * C, C)],
                    gsem.at[b],
                )

        def wait_gathers(b):
            pltpu.make_async_copy(
                out_hbm.at[pl.ds(0, CS), pl.ds(0, D)], rows_v.at[b], gsem.at[b]
            ).wait()

        def start_out(s, b):
            pltpu.async_copy(
                rows_v.at[b],
                out_hbm.at[pl.ds(base + s * CS, CS), pl.ds(0, D)],
                osem.at[b],
            )

        def wait_out(b):
            pltpu.make_async_copy(
                rows_v.at[b], out_hbm.at[pl.ds(0, CS), pl.ds(0, D)], osem.at[b]
            ).wait()

        fire_gathers(0, 0)

        @pl.loop(0, n_super)
        def _super(g):
            b = g % 2
            nb = (g + 1) % 2

            @pl.when(g + 1 < n_super)
            def _fire_next():
                @pl.when(g >= 1)
                def _recycle():
                    wait_out(nb)

                fire_gathers(g + 1, nb)

            wait_gathers(b)
            start_out(g, b)

        wait_out((n_super - 1) % 2)

    return k


@jax.jit
def kernel(ids, weight):
    S0, S1 = ids.shape
    B = S0 * S1
    ids3 = ids.astype(jnp.int32).reshape(NW, B // (NW * C), C)
    out = _make_kernel(B)(ids3, weight)
    return out.reshape(S0, S1, 2 * D)[..., :D]


# speedup vs baseline: 2.0348x; 1.3747x over previous
import functools

import jax
import jax.numpy as jnp
from jax import lax
from jax.experimental import pallas as pl
from jax.experimental.pallas import tpu as pltpu
from jax.experimental.pallas import tpu_sc as plsc

NC = 2
NS = 16
NW = NC * NS
D = 64
C = 128
K = 4
CS = C * K
NB = 2048


def _repack_body(xa_ref, xb_ref, o_ref):
    o_ref[:, 0:64] = xa_ref[...].T
    o_ref[:, 64:128] = xb_ref[...].T


def _repack(wT, V):
    ngrid = pl.cdiv(V, 2 * NB)
    last = V // NB
    return pl.pallas_call(
        _repack_body,
        out_shape=jax.ShapeDtypeStruct((ngrid * NB, 128), jnp.float32),
        grid=(ngrid,),
        in_specs=[
            pl.BlockSpec((64, NB), lambda j: (0, jnp.minimum(2 * j, last))),
            pl.BlockSpec((64, NB), lambda j: (0, jnp.minimum(2 * j + 1, last))),
        ],
        out_specs=pl.BlockSpec((NB, 128), lambda j: (j, 0)),
    )(wT, wT)


def _remap(r):
    j = r // (2 * NB)
    c = r % (2 * NB)
    return 2 * (j * NB + c % NB) + c // NB


@functools.lru_cache(maxsize=None)
def _make_kernel(B: int):
    n_chunks = B // (NW * C)
    n_super = B // (NW * CS)
    mesh = plsc.VectorSubcoreMesh(core_axis_name="c", subcore_axis_name="s")

    @functools.partial(
        pl.kernel,
        out_type=jax.ShapeDtypeStruct((B, 2 * D), jnp.float32),
        mesh=mesh,
        compiler_params=pltpu.CompilerParams(use_tc_tiling_on_sc=False),
        scratch_types=[
            pltpu.VMEM((n_chunks, C), jnp.int32),
            pltpu.VMEM((2, CS, D), jnp.float32),
            pltpu.SemaphoreType.DMA((2,)),
            pltpu.SemaphoreType.DMA((2,)),
        ],
    )
    def k(ids_hbm, w_hbm, out_hbm, idx_v, rows_v, gsem, osem):
        wid = lax.axis_index("s") * NC + lax.axis_index("c")
        base = wid * (n_super * CS)
        pltpu.sync_copy(ids_hbm.at[wid], idx_v)

        def fire_gathers(s, b):
            for j in range(K):
                pltpu.async_copy(
                    w_hbm.at[idx_v.at[s * K + j]],
                    rows_v.at[b, pl.ds(j * C, C)],
                    gsem.at[b],
                )

        def wait_gathers(b):
            pltpu.make_async_copy(
                out_hbm.at[pl.ds(0, CS), pl.ds(0, D)], rows_v.at[b], gsem.at[b]
            ).wait()

        def start_out(s, b):
            pltpu.async_copy(
                rows_v.at[b],
                out_hbm.at[pl.ds(base + s * CS, CS), pl.ds(0, D)],
                osem.at[b],
            )

        def wait_out(b):
            pltpu.make_async_copy(
                rows_v.at[b], out_hbm.at[pl.ds(0, CS), pl.ds(0, D)], osem.at[b]
            ).wait()

        fire_gathers(0, 0)

        @pl.loop(0, n_super)
        def _super(g):
            b = g % 2
            nb = (g + 1) % 2

            @pl.when(g + 1 < n_super)
            def _fire_next():
                @pl.when(g >= 1)
                def _recycle():
                    wait_out(nb)

                fire_gathers(g + 1, nb)

            wait_gathers(b)
            start_out(g, b)

        wait_out((n_super - 1) % 2)

    return k


@jax.jit
def kernel(ids, weight):
    S0, S1 = ids.shape
    B = S0 * S1
    ids2 = _remap(ids.astype(jnp.int32))
    ids3 = ids2.reshape(NW, B // (NW * C), C)
    V = weight.shape[0]
    w2 = _repack(weight.T, V)
    w_lin = w2.reshape(w2.shape[0] * 2, D)
    out = _make_kernel(B)(ids3, w_lin)
    return out.reshape(S0, S1, 2 * D)[..., :D]


# speedup vs baseline: 2.3872x; 1.1732x over previous
import functools

import jax
import jax.numpy as jnp
from jax import lax
from jax.experimental import pallas as pl
from jax.experimental.pallas import tpu as pltpu
from jax.experimental.pallas import tpu_sc as plsc

NC = 2
NS = 16
NW = NC * NS
D = 64
C = 128
K = 4
CS = C * K
NB = 8192


def _repack_body(xa_ref, xb_ref, o_ref):
    o_ref[:, 0:64] = xa_ref[...].T
    o_ref[:, 64:128] = xb_ref[...].T


def _repack(wT, V):
    ngrid = pl.cdiv(V, 2 * NB)
    last = V // NB
    return pl.pallas_call(
        _repack_body,
        out_shape=jax.ShapeDtypeStruct((ngrid * NB, 128), jnp.float32),
        grid=(ngrid,),
        in_specs=[
            pl.BlockSpec((64, NB), lambda j: (0, jnp.minimum(2 * j, last))),
            pl.BlockSpec((64, NB), lambda j: (0, jnp.minimum(2 * j + 1, last))),
        ],
        out_specs=pl.BlockSpec((NB, 128), lambda j: (j, 0)),
    )(wT, wT)


def _remap(r):
    j = r // (2 * NB)
    c = r % (2 * NB)
    return 2 * (j * NB + c % NB) + c // NB


@functools.lru_cache(maxsize=None)
def _make_kernel(B: int):
    n_chunks = B // (NW * C)
    n_super = B // (NW * CS)
    mesh = plsc.VectorSubcoreMesh(core_axis_name="c", subcore_axis_name="s")

    @functools.partial(
        pl.kernel,
        out_type=jax.ShapeDtypeStruct((B, 2 * D), jnp.float32),
        mesh=mesh,
        compiler_params=pltpu.CompilerParams(use_tc_tiling_on_sc=False),
        scratch_types=[
            pltpu.VMEM((n_chunks, C), jnp.int32),
            pltpu.VMEM((2, CS, D), jnp.float32),
            pltpu.SemaphoreType.DMA((2,)),
            pltpu.SemaphoreType.DMA((2,)),
        ],
    )
    def k(ids_hbm, w_hbm, out_hbm, idx_v, rows_v, gsem, osem):
        wid = lax.axis_index("s") * NC + lax.axis_index("c")
        base = wid * (n_super * CS)
        pltpu.sync_copy(ids_hbm.at[wid], idx_v)

        def fire_gathers(s, b):
            for j in range(K):
                pltpu.async_copy(
                    w_hbm.at[idx_v.at[s * K + j]],
                    rows_v.at[b, pl.ds(j * C, C)],
                    gsem.at[b],
                )

        def wait_gathers(b):
            pltpu.make_async_copy(
                out_hbm.at[pl.ds(0, CS), pl.ds(0, D)], rows_v.at[b], gsem.at[b]
            ).wait()

        def start_out(s, b):
            pltpu.async_copy(
                rows_v.at[b],
                out_hbm.at[pl.ds(base + s * CS, CS), pl.ds(0, D)],
                osem.at[b],
            )

        def wait_out(b):
            pltpu.make_async_copy(
                rows_v.at[b], out_hbm.at[pl.ds(0, CS), pl.ds(0, D)], osem.at[b]
            ).wait()

        fire_gathers(0, 0)

        @pl.loop(0, n_super)
        def _super(g):
            b = g % 2
            nb = (g + 1) % 2

            @pl.when(g + 1 < n_super)
            def _fire_next():
                @pl.when(g >= 1)
                def _recycle():
                    wait_out(nb)

                fire_gathers(g + 1, nb)

            wait_gathers(b)
            start_out(g, b)

        wait_out((n_super - 1) % 2)

    return k


@jax.jit
def kernel(ids, weight):
    S0, S1 = ids.shape
    B = S0 * S1
    ids2 = _remap(ids.astype(jnp.int32))
    ids3 = ids2.reshape(NW, B // (NW * C), C)
    V = weight.shape[0]
    w2 = _repack(weight.T, V)
    w_lin = w2.reshape(w2.shape[0] * 2, D)
    out = _make_kernel(B)(ids3, w_lin)
    return out.reshape(S0, S1, 2 * D)[..., :D]


# speedup vs baseline: 2.4450x; 1.0242x over previous
import functools

import jax
import jax.numpy as jnp
from jax import lax
from jax.experimental import pallas as pl
from jax.experimental.pallas import tpu as pltpu
from jax.experimental.pallas import tpu_sc as plsc

NC = 2
NS = 16
NW = NC * NS
D = 64
C = 128
K = 4
CS = C * K
NB = 16384


def _repack_body(xa_ref, xb_ref, o_ref):
    o_ref[:, 0:64] = xa_ref[...].T
    o_ref[:, 64:128] = xb_ref[...].T


def _repack(wT, V):
    ngrid = pl.cdiv(V, 2 * NB)
    last = V // NB
    return pl.pallas_call(
        _repack_body,
        out_shape=jax.ShapeDtypeStruct((ngrid * NB, 128), jnp.float32),
        grid=(ngrid,),
        in_specs=[
            pl.BlockSpec((64, NB), lambda j: (0, jnp.minimum(2 * j, last))),
            pl.BlockSpec((64, NB), lambda j: (0, jnp.minimum(2 * j + 1, last))),
        ],
        out_specs=pl.BlockSpec((NB, 128), lambda j: (j, 0)),
    )(wT, wT)


def _remap(r):
    j = r // (2 * NB)
    c = r % (2 * NB)
    return 2 * (j * NB + c % NB) + c // NB


@functools.lru_cache(maxsize=None)
def _make_kernel(B: int):
    n_chunks = B // (NW * C)
    n_super = B // (NW * CS)
    mesh = plsc.VectorSubcoreMesh(core_axis_name="c", subcore_axis_name="s")

    @functools.partial(
        pl.kernel,
        out_type=jax.ShapeDtypeStruct((B, 2 * D), jnp.float32),
        mesh=mesh,
        compiler_params=pltpu.CompilerParams(use_tc_tiling_on_sc=False),
        scratch_types=[
            pltpu.VMEM((n_chunks, C), jnp.int32),
            pltpu.VMEM((2, CS, D), jnp.float32),
            pltpu.SemaphoreType.DMA((2,)),
            pltpu.SemaphoreType.DMA((2,)),
        ],
    )
    def k(ids_hbm, w_hbm, out_hbm, idx_v, rows_v, gsem, osem):
        wid = lax.axis_index("s") * NC + lax.axis_index("c")
        base = wid * (n_super * CS)
        pltpu.sync_copy(ids_hbm.at[wid], idx_v)

        def fire_gathers(s, b):
            for j in range(K):
                pltpu.async_copy(
                    w_hbm.at[idx_v.at[s * K + j]],
                    rows_v.at[b, pl.ds(j * C, C)],
                    gsem.at[b],
                )

        def wait_gathers(b):
            pltpu.make_async_copy(
                out_hbm.at[pl.ds(0, CS), pl.ds(0, D)], rows_v.at[b], gsem.at[b]
            ).wait()

        def start_out(s, b):
            pltpu.async_copy(
                rows_v.at[b],
                out_hbm.at[pl.ds(base + s * CS, CS), pl.ds(0, D)],
                osem.at[b],
            )

        def wait_out(b):
            pltpu.make_async_copy(
                rows_v.at[b], out_hbm.at[pl.ds(0, CS), pl.ds(0, D)], osem.at[b]
            ).wait()

        fire_gathers(0, 0)

        @pl.loop(0, n_super)
        def _super(g):
            b = g % 2
            nb = (g + 1) % 2

            @pl.when(g + 1 < n_super)
            def _fire_next():
                @pl.when(g >= 1)
                def _recycle():
                    wait_out(nb)

                fire_gathers(g + 1, nb)

            wait_gathers(b)
            start_out(g, b)

        wait_out((n_super - 1) % 2)

    return k


@jax.jit
def kernel(ids, weight):
    S0, S1 = ids.shape
    B = S0 * S1
    ids2 = _remap(ids.astype(jnp.int32))
    ids3 = ids2.reshape(NW, B // (NW * C), C)
    V = weight.shape[0]
    w2 = _repack(weight.T, V)
    w_lin = w2.reshape(w2.shape[0] * 2, D)
    out = _make_kernel(B)(ids3, w_lin)
    return out.reshape(S0, S1, 2 * D)[..., :D]
